# baseline (speedup 1.0000x reference)
import jax
import jax.numpy as jnp
from jax import lax
from jax.experimental import pallas as pl
from jax.experimental.pallas import tpu as pltpu
from jax.experimental.pallas import tpu_sc as plsc

NC = 2
NS = 16
NW = NC * NS
L = 16
CP = 8


def _pick_block(ew: int) -> int:
    best = L
    for b in range(L, 801, L):
        if ew % b == 0 and (ew // b) % 2 == 0:
            best = b
    return best


def kernel(charges, cell, positions, neighbor_indices, neighbor_distances):
    n, c = charges.shape
    e = neighbor_indices.shape[0]
    assert e % NW == 0, e
    ew = e // NW
    blk = _pick_block(ew)
    npair = ew // blk // 2

    mesh = plsc.VectorSubcoreMesh(
        core_axis_name="c", subcore_axis_name="s", num_cores=NC, num_subcores=NS)

    def body(q_hbm, ij_hbm, ndist_hbm, zeros_hbm, out_hbm,
             q_sp, acc_sp,
             w0, ij0, rg0,
             w1, ij1, rg1,
             slin0, slin1, sg0, sg1, ss0, ss1):
        cid = lax.axis_index("c")
        sid = lax.axis_index("s")
        wid = cid * NS + sid

        chunk = n // NS
        rem = n - NS * chunk
        srows = pl.ds(sid * chunk, chunk)
        pltpu.sync_copy(q_hbm.at[srows], q_sp.at[srows])
        pltpu.sync_copy(zeros_hbm.at[srows], acc_sp.at[srows])
        if rem:
            trows = pl.ds(NS * chunk, rem)

            @pl.when(sid == 0)
            def _tail():
                pltpu.sync_copy(q_hbm.at[trows], q_sp.at[trows])
                pltpu.sync_copy(zeros_hbm.at[trows], acc_sp.at[trows])

        plsc.subcore_barrier()

        iota = lax.iota(jnp.int32, L)
        pat = iota // CP
        colpat = iota - pat * CP
        swpat = iota ^ 1
        base = wid * ew

        def linload(off, wb, ijb, sem):
            pltpu.async_copy(ij_hbm.at[pl.ds(2 * off, 2 * blk)], ijb, sem)
            pltpu.async_copy(ndist_hbm.at[pl.ds(off, blk)], wb, sem)

        def linwait(off, wb, ijb, sem):
            pltpu.make_async_copy(
                ij_hbm.at[pl.ds(2 * off, 2 * blk)], ijb, sem).wait()
            pltpu.make_async_copy(
                ndist_hbm.at[pl.ds(off, blk)], wb, sem).wait()

        def swap(ijb):
            @plsc.parallel_loop(0, (2 * blk) // L, unroll=4)
            def _(g):
                s = g * L
                ijb[pl.ds(s, L)] = plsc.load_gather(ijb, [s + swpat])

        def wscale(wb):
            @plsc.parallel_loop(0, blk // L, unroll=4)
            def _(g):
                s = g * L
                wb[pl.ds(s, L)] = 0.5 / wb[pl.ds(s, L)]

        def scale(wb, rgb):
            @plsc.parallel_loop(0, blk // 2, unroll=4)
            def _(g):
                wrow = 2 * g + pat
                w16 = plsc.load_gather(wb, [wrow])
                re = 4 * g + 2 * pat
                va = plsc.load_gather(rgb, [re, colpat]) * w16
                plsc.store_scatter(rgb, [re, colpat], va)
                ro = re + 1
                vb = plsc.load_gather(rgb, [ro, colpat]) * w16
                plsc.store_scatter(rgb, [ro, colpat], vb)

        def phase(off, wb, ijb, rgb, slin, sg):
            linwait(off, wb, ijb, slin)
            pltpu.async_copy(q_sp.at[ijb], rgb, sg)
            wscale(wb)
            pltpu.make_async_copy(q_sp.at[ijb], rgb, sg).wait()
            swap(ijb)
            scale(wb, rgb)

        def scatter(rgb, ijb, ss):
            pltpu.async_copy(rgb, acc_sp.at[ijb], ss, add=True)

        def scatwait(rgb, ijb, ss):
            pltpu.make_async_copy(rgb, acc_sp.at[ijb], ss).wait()

        linload(base, w0, ij0, slin0)

        def pair(b2, carry):
            off0 = base + (2 * b2) * blk
            off1 = off0 + blk
            phase(off0, w0, ij0, rg0, slin0, sg0)

            @pl.when(b2 >= 1)
            def _():
                scatwait(rg1, ij1, ss1)

            scatter(rg0, ij0, ss0)
            linload(off1, w1, ij1, slin1)

            phase(off1, w1, ij1, rg1, slin1, sg1)
            scatter(rg1, ij1, ss1)
            scatwait(rg0, ij0, ss0)

            @pl.when(b2 + 1 < npair)
            def _():
                linload(base + (2 * b2 + 2) * blk, w0, ij0, slin0)

            return carry

        lax.fori_loop(0, npair, pair, 0)
        scatwait(rg1, ij1, ss1)

        plsc.subcore_barrier()

        pltpu.sync_copy(acc_sp.at[srows],
                        out_hbm.at[pl.ds(cid * n + sid * chunk, chunk)])
        if rem:
            @pl.when(sid == 0)
            def _wtail():
                pltpu.sync_copy(acc_sp.at[pl.ds(NS * chunk, rem)],
                                out_hbm.at[pl.ds(cid * n + NS * chunk, rem)])

    kfn = pl.kernel(
        body,
        out_type=jax.ShapeDtypeStruct((NC * n, CP), jnp.float32),
        mesh=mesh,
        compiler_params=pltpu.CompilerParams(
            needs_layout_passes=False, use_tc_tiling_on_sc=False),
        scratch_types=[
            pltpu.VMEM_SHARED((n, CP), jnp.float32),
            pltpu.VMEM_SHARED((n, CP), jnp.float32),
            pltpu.VMEM((blk,), jnp.float32),
            pltpu.VMEM((2 * blk,), jnp.int32),
            pltpu.VMEM((2 * blk, CP), jnp.float32),
            pltpu.VMEM((blk,), jnp.float32),
            pltpu.VMEM((2 * blk,), jnp.int32),
            pltpu.VMEM((2 * blk, CP), jnp.float32),
            pltpu.SemaphoreType.DMA,
            pltpu.SemaphoreType.DMA,
            pltpu.SemaphoreType.DMA,
            pltpu.SemaphoreType.DMA,
            pltpu.SemaphoreType.DMA,
            pltpu.SemaphoreType.DMA,
        ],
    )

    qpad = jnp.pad(charges, ((0, 0), (0, CP - c)))
    zeros = jnp.zeros((n, CP), jnp.float32)
    partial = kfn(qpad, neighbor_indices.reshape(-1),
                  neighbor_distances, zeros)
    return partial[:n, :c] + partial[n:, :c]


# speedup vs baseline: 10.9418x; 10.9418x over previous
import jax
import jax.numpy as jnp
from jax import lax
from jax.experimental import pallas as pl
from jax.experimental.pallas import tpu as pltpu
from jax.experimental.pallas import tpu_sc as plsc

NC = 2
NS = 16
NW = NC * NS
L = 16
CP = 8


def _pick_block(ew: int) -> int:
    best = L
    for b in range(L, 801, L):
        if ew % b == 0 and (ew // b) % 2 == 0:
            best = b
    return best


def kernel(charges, cell, positions, neighbor_indices, neighbor_distances):
    n, c = charges.shape
    e = neighbor_indices.shape[0]
    assert e % NW == 0, e
    ew = e // NW
    blk = _pick_block(ew)
    npair = ew // blk // 2

    mesh = plsc.VectorSubcoreMesh(
        core_axis_name="c", subcore_axis_name="s", num_cores=NC, num_subcores=NS)

    def body(q_hbm, ii_hbm, jj_hbm, ndist_hbm, zeros_hbm, out_hbm,
             q_sp, acc_sp,
             w0, ia0, ja0, ra0, rb0,
             w1, ia1, ja1, ra1, rb1,
             slin0, slin1, sg0, sg1, ss0, ss1):
        cid = lax.axis_index("c")
        sid = lax.axis_index("s")
        wid = cid * NS + sid

        @pl.when(sid == 0)
        def _stage():
            pltpu.sync_copy(q_hbm, q_sp)
            pltpu.sync_copy(zeros_hbm, acc_sp)

        plsc.subcore_barrier()

        iota = lax.iota(jnp.int32, L)
        pat = iota // CP
        colpat = iota - pat * CP
        base = wid * ew

        def linload(off, wb, iab, jab, sem):
            pltpu.async_copy(ii_hbm.at[pl.ds(off, blk)], iab, sem)
            pltpu.async_copy(jj_hbm.at[pl.ds(off, blk)], jab, sem)
            pltpu.async_copy(ndist_hbm.at[pl.ds(off, blk)], wb, sem)

        def linwait(off, wb, iab, jab, sem):
            pltpu.make_async_copy(ii_hbm.at[pl.ds(off, blk)], iab, sem).wait()
            pltpu.make_async_copy(jj_hbm.at[pl.ds(off, blk)], jab, sem).wait()
            pltpu.make_async_copy(ndist_hbm.at[pl.ds(off, blk)], wb, sem).wait()

        def wscale(wb):
            @plsc.parallel_loop(0, blk // L, unroll=4)
            def _(g):
                s = g * L
                wb[pl.ds(s, L)] = 0.5 / wb[pl.ds(s, L)]

        def scale(wb, rab, rbb):
            @plsc.parallel_loop(0, (blk * CP) // L, unroll=4)
            def _(g):
                r = g * (L // CP)
                row_idx = r + pat
                w16 = plsc.load_gather(wb, [row_idx])
                va = plsc.load_gather(rab, [row_idx, colpat]) * w16
                plsc.store_scatter(rab, [row_idx, colpat], va)
                vb = plsc.load_gather(rbb, [row_idx, colpat]) * w16
                plsc.store_scatter(rbb, [row_idx, colpat], vb)

        def phase(off, wb, iab, jab, rab, rbb, slin, sg):
            linwait(off, wb, iab, jab, slin)
            pltpu.async_copy(q_sp.at[jab], rab, sg)
            pltpu.async_copy(q_sp.at[iab], rbb, sg)
            wscale(wb)
            pltpu.make_async_copy(q_sp.at[jab], rab, sg).wait()
            pltpu.make_async_copy(q_sp.at[iab], rbb, sg).wait()
            scale(wb, rab, rbb)

        def scatter(rab, rbb, iab, jab, ss):
            pltpu.async_copy(rab, acc_sp.at[iab], ss, add=True)
            pltpu.async_copy(rbb, acc_sp.at[jab], ss, add=True)

        def scatwait(rab, rbb, iab, jab, ss):
            pltpu.make_async_copy(rab, acc_sp.at[iab], ss).wait()
            pltpu.make_async_copy(rbb, acc_sp.at[jab], ss).wait()

        linload(base, w0, ia0, ja0, slin0)

        def pair(b2, carry):
            off0 = base + (2 * b2) * blk
            off1 = off0 + blk
            phase(off0, w0, ia0, ja0, ra0, rb0, slin0, sg0)

            @pl.when(b2 >= 1)
            def _():
                scatwait(ra1, rb1, ia1, ja1, ss1)

            scatter(ra0, rb0, ia0, ja0, ss0)
            linload(off1, w1, ia1, ja1, slin1)

            phase(off1, w1, ia1, ja1, ra1, rb1, slin1, sg1)
            scatter(ra1, rb1, ia1, ja1, ss1)
            scatwait(ra0, rb0, ia0, ja0, ss0)

            @pl.when(b2 + 1 < npair)
            def _():
                linload(base + (2 * b2 + 2) * blk, w0, ia0, ja0, slin0)

            return carry

        lax.fori_loop(0, npair, pair, 0)
        scatwait(ra1, rb1, ia1, ja1, ss1)

        plsc.subcore_barrier()

        @pl.when(sid == 0)
        def _writeout():
            pltpu.sync_copy(acc_sp, out_hbm.at[pl.ds(cid * n, n)])

    kfn = pl.kernel(
        body,
        out_type=jax.ShapeDtypeStruct((NC * n, CP), jnp.float32),
        mesh=mesh,
        compiler_params=pltpu.CompilerParams(
            needs_layout_passes=False, use_tc_tiling_on_sc=False),
        scratch_types=[
            pltpu.VMEM_SHARED((n, CP), jnp.float32),
            pltpu.VMEM_SHARED((n, CP), jnp.float32),
            pltpu.VMEM((blk,), jnp.float32),
            pltpu.VMEM((blk,), jnp.int32),
            pltpu.VMEM((blk,), jnp.int32),
            pltpu.VMEM((blk, CP), jnp.float32),
            pltpu.VMEM((blk, CP), jnp.float32),
            pltpu.VMEM((blk,), jnp.float32),
            pltpu.VMEM((blk,), jnp.int32),
            pltpu.VMEM((blk,), jnp.int32),
            pltpu.VMEM((blk, CP), jnp.float32),
            pltpu.VMEM((blk, CP), jnp.float32),
            pltpu.SemaphoreType.DMA,
            pltpu.SemaphoreType.DMA,
            pltpu.SemaphoreType.DMA,
            pltpu.SemaphoreType.DMA,
            pltpu.SemaphoreType.DMA,
            pltpu.SemaphoreType.DMA,
        ],
    )

    qpad = jnp.pad(charges, ((0, 0), (0, CP - c)))
    zeros = jnp.zeros((n, CP), jnp.float32)
    partial = kfn(qpad, neighbor_indices[:, 0], neighbor_indices[:, 1],
                  neighbor_distances, zeros)
    return partial[:n, :c] + partial[n:, :c]
